# baseline (speedup 1.0000x reference)
import jax
import jax.numpy as jnp
from jax.experimental import pallas as pl
from jax.experimental.pallas import tpu as pltpu

NBINS = 100
BATCHES_PER_BLOCK = 4


def _norm_body(t_ref, mean_ref, std_ref, x_ref, o_ref, *, rows_per_batch, nb):
    i = pl.program_id(0)
    for j in range(nb):
        b = i * nb + j
        tv = t_ref[b]
        bin_ = jnp.clip((tv * NBINS).astype(jnp.int32), 0, NBINS - 1)
        m = mean_ref[bin_]
        inv = 1.0 / std_ref[bin_]
        sl = pl.ds(j * rows_per_batch, rows_per_batch)
        o_ref[sl, :] = (x_ref[sl, :] - m) * inv


def kernel(x_t, t, data_mean, data_std):
    B, C, H, W = x_t.shape
    rows_per_batch = C * H
    x2 = x_t.reshape(B * C * H, W)

    nb = BATCHES_PER_BLOCK
    rows_blk = nb * rows_per_batch
    grid = (B // nb,)
    import functools
    body = functools.partial(_norm_body, rows_per_batch=rows_per_batch, nb=nb)
    out = pl.pallas_call(
        body,
        grid=grid,
        in_specs=[
            pl.BlockSpec(memory_space=pltpu.SMEM),
            pl.BlockSpec(memory_space=pltpu.SMEM),
            pl.BlockSpec(memory_space=pltpu.SMEM),
            pl.BlockSpec((rows_blk, W), lambda i: (i, 0)),
        ],
        out_specs=pl.BlockSpec((rows_blk, W), lambda i: (i, 0)),
        out_shape=jax.ShapeDtypeStruct((B * C * H, W), jnp.float32),
    )(t, data_mean, data_std, x2)
    return out.reshape(B, C, H, W)


# speedup vs baseline: 6.2926x; 6.2926x over previous
import jax
import jax.numpy as jnp
from jax.experimental import pallas as pl

NBINS = 100
ROWS_PER_STEP = 2048


def _norm_body(t_ref, mean_ref, std_ref, x_ref, o_ref):
    tr = t_ref[...]
    bins = jnp.clip((tr * NBINS).astype(jnp.int32), 0, NBINS - 1)
    krows = jax.lax.broadcasted_iota(jnp.int32, (NBINS, 1), 0)
    oh = (krows == bins).astype(jnp.float32)
    m = jnp.dot(mean_ref[...], oh, preferred_element_type=jnp.float32)
    s = jnp.dot(std_ref[...], oh, preferred_element_type=jnp.float32)
    o_ref[...] = (x_ref[...] - m) * (1.0 / s)


def kernel(x_t, t, data_mean, data_std):
    B, C, H, W = x_t.shape
    F = C * H * W
    xv = jnp.transpose(x_t, (1, 2, 3, 0)).reshape(F, B)
    t_row = t.reshape(1, B)
    mean_row = data_mean.reshape(1, NBINS)
    std_row = data_std.reshape(1, NBINS)

    S = ROWS_PER_STEP
    grid = (F // S,)
    out = pl.pallas_call(
        _norm_body,
        grid=grid,
        in_specs=[
            pl.BlockSpec((1, B), lambda i: (0, 0)),
            pl.BlockSpec((1, NBINS), lambda i: (0, 0)),
            pl.BlockSpec((1, NBINS), lambda i: (0, 0)),
            pl.BlockSpec((S, B), lambda i: (i, 0)),
        ],
        out_specs=pl.BlockSpec((S, B), lambda i: (i, 0)),
        out_shape=jax.ShapeDtypeStruct((F, B), jnp.float32),
    )(t_row, mean_row, std_row, xv)
    return jnp.transpose(out.reshape(C, H, W, B), (3, 0, 1, 2))
